# Initial kernel scaffold; baseline (speedup 1.0000x reference)
import functools

import jax
import jax.numpy as jnp
from jax import lax
from jax.experimental import pallas as pl
from jax.experimental.pallas import tpu as pltpu
from jax.experimental.pallas import tpu_sc as plsc

N = 10000
E = 320000
HID = 20
L = 4

N_EG = 8
N_DG = 4
DPG = HID // N_DG
EPW = E // N_EG
CHUNK = 2000
NCHUNKS = EPW // CHUNK
NGROUPS = CHUNK // 16
TBL = N * DPG
TBL_PAD = TBL + 16

DEG_EPW = E // 32
DEG_GROUPS = DEG_EPW // 16


def _wid():
    return lax.axis_index("c") * 16 + lax.axis_index("s")


def _sc_mesh():
    return plsc.VectorSubcoreMesh(core_axis_name="c", subcore_axis_name="s")


def _deg_body(col_hbm, ew_hbm, out_hbm, cbuf, wbuf, acc):
    wid = _wid()
    base = wid * DEG_EPW
    pltpu.sync_copy(col_hbm.at[pl.ds(base, DEG_EPW)], cbuf)
    pltpu.sync_copy(ew_hbm.at[pl.ds(base, DEG_EPW)], wbuf)

    def zero(i, _):
        acc[pl.ds(i * 16, 16)] = jnp.zeros((16,), jnp.float32)
        return 0

    lax.fori_loop(0, (N + 16) // 16, zero, 0)

    def group(g, _):
        cc = cbuf[pl.ds(g * 16, 16)]
        w = wbuf[pl.ds(g * 16, 16)]
        occ, _last = plsc.scan_count(cc)
        plsc.addupdate_scatter(acc, [cc], w, mask=occ == 0)
        mx = jnp.max(occ)

        @pl.when(mx > 0)
        def _():
            def dup_pass(k, _c):
                plsc.addupdate_scatter(acc, [cc], w, mask=occ == k)
                return 0

            lax.fori_loop(1, mx + 1, dup_pass, 0)

        return 0

    lax.fori_loop(0, DEG_GROUPS, group, 0)
    pltpu.sync_copy(acc.at[pl.ds(0, N)], out_hbm.at[wid])


_deg_call = pl.kernel(
    _deg_body,
    out_type=jax.ShapeDtypeStruct((32, N), jnp.float32),
    mesh=_sc_mesh(),
    scratch_types=[
        pltpu.VMEM((DEG_EPW,), jnp.int32),
        pltpu.VMEM((DEG_EPW,), jnp.float32),
        pltpu.VMEM((N + 16,), jnp.float32),
    ],
)


def _edge_body(hp_hbm, row_hbm, col_hbm, ew_hbm, out_hbm,
               tbl, acc, rbuf, cbuf, wbuf):
    wid = _wid()
    eg = wid // N_DG
    dg = wid % N_DG

    pltpu.sync_copy(hp_hbm.at[pl.ds(dg * TBL, TBL)], tbl.at[pl.ds(0, TBL)])

    def zero(i, _):
        acc[pl.ds(i * 16, 16)] = jnp.zeros((16,), jnp.float32)
        return 0

    lax.fori_loop(0, TBL_PAD // 16, zero, 0)

    def chunk(ch, _):
        base = eg * EPW + ch * CHUNK
        pltpu.sync_copy(row_hbm.at[pl.ds(base, CHUNK)], rbuf)
        pltpu.sync_copy(col_hbm.at[pl.ds(base, CHUNK)], cbuf)
        pltpu.sync_copy(ew_hbm.at[pl.ds(base, CHUNK)], wbuf)

        def group(g, _c):
            r = rbuf[pl.ds(g * 16, 16)]
            cc = cbuf[pl.ds(g * 16, 16)]
            w = wbuf[pl.ds(g * 16, 16)]
            occ, _last = plsc.scan_count(cc)
            first = occ == 0
            for d in range(DPG):
                v = plsc.load_gather(tbl, [r + d * N])
                plsc.addupdate_scatter(acc, [cc + d * N], v * w, mask=first)
            mx = jnp.max(occ)

            @pl.when(mx > 0)
            def _():
                def dup_pass(k, _cc):
                    m = occ == k
                    for d in range(DPG):
                        v = plsc.load_gather(tbl, [r + d * N])
                        plsc.addupdate_scatter(acc, [cc + d * N], v * w,
                                               mask=m)
                    return 0

                lax.fori_loop(1, mx + 1, dup_pass, 0)

            return 0

        lax.fori_loop(0, NGROUPS, group, 0)
        return 0

    lax.fori_loop(0, NCHUNKS, chunk, 0)
    pltpu.sync_copy(acc.at[pl.ds(0, TBL)], out_hbm.at[eg, dg])


_edge_call = pl.kernel(
    _edge_body,
    out_type=jax.ShapeDtypeStruct((N_EG, N_DG, TBL), jnp.float32),
    mesh=_sc_mesh(),
    scratch_types=[
        pltpu.VMEM((TBL_PAD,), jnp.float32),
        pltpu.VMEM((TBL_PAD,), jnp.float32),
        pltpu.VMEM((CHUNK,), jnp.int32),
        pltpu.VMEM((CHUNK,), jnp.int32),
        pltpu.VMEM((CHUNK,), jnp.float32),
    ],
)


def _tc1_body(deg_parts_ref, ft_ref, w0t_ref, dinv_ref, h0_ref, hp0_ref):
    deg = jnp.sum(deg_parts_ref[...], axis=0, keepdims=True) + 1.0
    dinv = 1.0 / jnp.sqrt(deg)
    dinv_ref[...] = dinv
    h0 = jnp.dot(w0t_ref[...], ft_ref[...], preferred_element_type=jnp.float32)
    h0_ref[...] = h0
    hp0_ref[...] = h0 * dinv


_tc1_call = pl.pallas_call(
    _tc1_body,
    out_shape=(
        jax.ShapeDtypeStruct((1, N), jnp.float32),
        jax.ShapeDtypeStruct((HID, N), jnp.float32),
        jax.ShapeDtypeStruct((HID, N), jnp.float32),
    ),
)


def _tc_layer_body(has_prev, has_next,
                   parts_ref, h_ref, dinv_ref, bias_ref, gam_ref, bta_ref,
                   *rest):
    rest = list(rest)
    prev_ref = rest.pop(0) if has_prev else None
    wt_ref = rest.pop(0) if has_next else None
    out_ref = rest.pop(0)
    h_next_ref = rest.pop(0) if has_next else None
    hp_next_ref = rest.pop(0) if has_next else None

    parts = parts_ref[...]
    acc = parts[0]
    for k in range(1, N_EG):
        acc = acc + parts[k]
    dinv = dinv_ref[...]
    agg = acc * dinv + h_ref[...] * (dinv * dinv) + bias_ref[...]
    m = jnp.mean(agg, axis=1, keepdims=True)
    cen = agg - m
    var = jnp.mean(cen * cen, axis=1, keepdims=True)
    x = cen / jnp.sqrt(var + 1e-5) * gam_ref[...] + bta_ref[...]
    x = jnp.maximum(x, 0.0)
    if has_prev:
        x = x + 0.7 * prev_ref[...]
    out_ref[...] = x
    if has_next:
        h_next = jnp.dot(wt_ref[...], x, preferred_element_type=jnp.float32)
        h_next_ref[...] = h_next
        hp_next_ref[...] = h_next * dinv


def _tc_layer_call(has_prev, has_next):
    n_out = 1 + (2 if has_next else 0)
    shapes = [jax.ShapeDtypeStruct((HID, N), jnp.float32)] * n_out
    return pl.pallas_call(
        functools.partial(_tc_layer_body, has_prev, has_next),
        out_shape=tuple(shapes) if n_out > 1 else shapes[0],
    )


def _tc_final_body(out0_ref, out1_ref, out2_ref, out3_ref, lw_ref, emb_ref):
    w = jax.nn.softmax(lw_ref[...], axis=-1)
    emb = (out0_ref[...] * w[0, 0] + out1_ref[...] * w[0, 1]
           + out2_ref[...] * w[0, 2] + out3_ref[...] * w[0, 3])
    emb_ref[...] = emb


_tc_final_call = pl.pallas_call(
    _tc_final_body,
    out_shape=jax.ShapeDtypeStruct((HID, N), jnp.float32),
)


def kernel(features, edges, edge_weight, W0, Wh, b, gamma, beta, lw):
    row = edges[0]
    col = edges[1]
    ft = features.T
    w0t = W0.T

    deg_parts = _deg_call(col, edge_weight)
    dinv, h, hp = _tc1_call(deg_parts, ft, w0t)

    outs = []
    prev = None
    for i in range(L):
        parts = _edge_call(hp.reshape(HID * N), row, col, edge_weight)
        parts = parts.reshape(N_EG, HID, N)
        has_next = i < L - 1
        args = [parts, h,
                dinv,
                b[i].reshape(HID, 1),
                gamma[i].reshape(HID, 1),
                beta[i].reshape(HID, 1)]
        if prev is not None:
            args.append(prev)
        if has_next:
            args.append(Wh[i].T)
        res = _tc_layer_call(prev is not None, has_next)(*args)
        if has_next:
            out, h, hp = res
        else:
            out = res
        outs.append(out)
        prev = out

    emb_t = _tc_final_call(outs[0], outs[1], outs[2], outs[3],
                           lw.reshape(1, L))
    return emb_t.T


# speedup vs baseline: 12.6121x; 12.6121x over previous
import functools

import jax
import jax.numpy as jnp
from jax import lax
from jax.experimental import pallas as pl
from jax.experimental.pallas import tpu as pltpu
from jax.experimental.pallas import tpu_sc as plsc

N = 10000
E = 320000
HID = 20
L = 4

N_EG = 8
N_DG = 4
DPG = HID // N_DG
EPW = E // N_EG
CHUNK = 2000
NCHUNKS = EPW // CHUNK
NGROUPS = CHUNK // 16
TBL = N * DPG
TBL_PAD = TBL + 16

DEG_EPW = E // 32
DEG_GROUPS = DEG_EPW // 16


def _wid():
    return lax.axis_index("c") * 16 + lax.axis_index("s")


def _sc_mesh():
    return plsc.VectorSubcoreMesh(core_axis_name="c", subcore_axis_name="s")


def _deg_body(col_hbm, ew_hbm, out_hbm, cbuf, wbuf, acc):
    wid = _wid()
    base = wid * DEG_EPW
    pltpu.sync_copy(col_hbm.at[pl.ds(base, DEG_EPW)], cbuf)
    pltpu.sync_copy(ew_hbm.at[pl.ds(base, DEG_EPW)], wbuf)

    def zero(i, _):
        acc[pl.ds(i * 16, 16)] = jnp.zeros((16,), jnp.float32)
        return 0

    lax.fori_loop(0, (N + 16) // 16, zero, 0)

    def group(g, _):
        cc = cbuf[pl.ds(g * 16, 16)]
        w = wbuf[pl.ds(g * 16, 16)]
        occ, _last = plsc.scan_count(cc)
        plsc.addupdate_scatter(acc, [cc], w, mask=occ == 0)
        mx = jnp.max(occ)

        @pl.when(mx > 0)
        def _():
            def dup_pass(k, _c):
                plsc.addupdate_scatter(acc, [cc], w, mask=occ == k)
                return 0

            lax.fori_loop(1, mx + 1, dup_pass, 0)

        return 0

    lax.fori_loop(0, DEG_GROUPS, group, 0)
    pltpu.sync_copy(acc.at[pl.ds(0, N)], out_hbm.at[pl.ds(wid * N, N)])


_deg_call = pl.kernel(
    _deg_body,
    out_type=jax.ShapeDtypeStruct((32 * N,), jnp.float32),
    mesh=_sc_mesh(),
    compiler_params=pltpu.CompilerParams(needs_layout_passes=False),
    scratch_types=[
        pltpu.VMEM((DEG_EPW,), jnp.int32),
        pltpu.VMEM((DEG_EPW,), jnp.float32),
        pltpu.VMEM((N + 16,), jnp.float32),
    ],
)


def _edge_body(hp_hbm, row_hbm, col_hbm, ew_hbm, out_hbm,
               tbl, acc, rbuf, cbuf, wbuf):
    wid = _wid()
    eg = wid // N_DG
    dg = wid % N_DG

    pltpu.sync_copy(hp_hbm.at[pl.ds(dg * TBL, TBL)], tbl.at[pl.ds(0, TBL)])

    def zero(i, _):
        acc[pl.ds(i * 16, 16)] = jnp.zeros((16,), jnp.float32)
        return 0

    lax.fori_loop(0, TBL_PAD // 16, zero, 0)

    def chunk(ch, _):
        base = eg * EPW + ch * CHUNK
        pltpu.sync_copy(row_hbm.at[pl.ds(base, CHUNK)], rbuf)
        pltpu.sync_copy(col_hbm.at[pl.ds(base, CHUNK)], cbuf)
        pltpu.sync_copy(ew_hbm.at[pl.ds(base, CHUNK)], wbuf)

        def group(g, _c):
            r = rbuf[pl.ds(g * 16, 16)]
            cc = cbuf[pl.ds(g * 16, 16)]
            w = wbuf[pl.ds(g * 16, 16)]
            occ, _last = plsc.scan_count(cc)
            first = occ == 0
            for d in range(DPG):
                v = plsc.load_gather(tbl, [r + d * N])
                plsc.addupdate_scatter(acc, [cc + d * N], v * w, mask=first)
            mx = jnp.max(occ)

            @pl.when(mx > 0)
            def _():
                def dup_pass(k, _cc):
                    m = occ == k
                    for d in range(DPG):
                        v = plsc.load_gather(tbl, [r + d * N])
                        plsc.addupdate_scatter(acc, [cc + d * N], v * w,
                                               mask=m)
                    return 0

                lax.fori_loop(1, mx + 1, dup_pass, 0)

            return 0

        lax.fori_loop(0, NGROUPS, group, 0)
        return 0

    lax.fori_loop(0, NCHUNKS, chunk, 0)
    pltpu.sync_copy(acc.at[pl.ds(0, TBL)], out_hbm.at[pl.ds(wid * TBL, TBL)])


_edge_call = pl.kernel(
    _edge_body,
    out_type=jax.ShapeDtypeStruct((N_EG * N_DG * TBL,), jnp.float32),
    mesh=_sc_mesh(),
    compiler_params=pltpu.CompilerParams(needs_layout_passes=False),
    scratch_types=[
        pltpu.VMEM((TBL_PAD,), jnp.float32),
        pltpu.VMEM((TBL_PAD,), jnp.float32),
        pltpu.VMEM((CHUNK,), jnp.int32),
        pltpu.VMEM((CHUNK,), jnp.int32),
        pltpu.VMEM((CHUNK,), jnp.float32),
    ],
)


def _tc1_body(deg_parts_ref, ft_ref, w0t_ref, dinv_ref, h0_ref, hp0_ref):
    deg = jnp.sum(deg_parts_ref[...], axis=0, keepdims=True) + 1.0
    dinv = 1.0 / jnp.sqrt(deg)
    dinv_ref[...] = dinv
    h0 = jnp.dot(w0t_ref[...], ft_ref[...], preferred_element_type=jnp.float32)
    h0_ref[...] = h0
    hp0_ref[...] = h0 * dinv


_tc1_call = pl.pallas_call(
    _tc1_body,
    out_shape=(
        jax.ShapeDtypeStruct((1, N), jnp.float32),
        jax.ShapeDtypeStruct((HID, N), jnp.float32),
        jax.ShapeDtypeStruct((HID, N), jnp.float32),
    ),
)


def _tc_layer_body(has_prev, has_next,
                   parts_ref, h_ref, dinv_ref, bias_ref, gam_ref, bta_ref,
                   *rest):
    rest = list(rest)
    prev_ref = rest.pop(0) if has_prev else None
    wt_ref = rest.pop(0) if has_next else None
    out_ref = rest.pop(0)
    h_next_ref = rest.pop(0) if has_next else None
    hp_next_ref = rest.pop(0) if has_next else None

    acc = parts_ref[0]
    for k in range(1, N_EG):
        acc = acc + parts_ref[k]
    dinv = dinv_ref[...]
    agg = acc * dinv + h_ref[...] * (dinv * dinv) + bias_ref[...]
    m = jnp.mean(agg, axis=1, keepdims=True)
    cen = agg - m
    var = jnp.mean(cen * cen, axis=1, keepdims=True)
    x = cen / jnp.sqrt(var + 1e-5) * gam_ref[...] + bta_ref[...]
    x = jnp.maximum(x, 0.0)
    if has_prev:
        x = x + 0.7 * prev_ref[...]
    out_ref[...] = x
    if has_next:
        h_next = jnp.dot(wt_ref[...], x, preferred_element_type=jnp.float32)
        h_next_ref[...] = h_next
        hp_next_ref[...] = h_next * dinv


def _tc_layer_call(has_prev, has_next):
    n_out = 1 + (2 if has_next else 0)
    shapes = [jax.ShapeDtypeStruct((HID, N), jnp.float32)] * n_out
    return pl.pallas_call(
        functools.partial(_tc_layer_body, has_prev, has_next),
        out_shape=tuple(shapes) if n_out > 1 else shapes[0],
    )


def _tc_final_body(out0_ref, out1_ref, out2_ref, out3_ref, lw_ref, emb_ref):
    w = jax.nn.softmax(lw_ref[...], axis=-1)
    emb = (out0_ref[...] * w[0, 0] + out1_ref[...] * w[0, 1]
           + out2_ref[...] * w[0, 2] + out3_ref[...] * w[0, 3])
    emb_ref[...] = emb


_tc_final_call = pl.pallas_call(
    _tc_final_body,
    out_shape=jax.ShapeDtypeStruct((HID, N), jnp.float32),
)


def kernel(features, edges, edge_weight, W0, Wh, b, gamma, beta, lw):
    row = edges[0]
    col = edges[1]
    ft = features.T
    w0t = W0.T

    deg_parts = _deg_call(col, edge_weight).reshape(32, N)
    dinv, h, hp = _tc1_call(deg_parts, ft, w0t)

    outs = []
    prev = None
    for i in range(L):
        parts = _edge_call(hp.reshape(HID * N), row, col, edge_weight)
        parts = parts.reshape(N_EG, HID, N)
        has_next = i < L - 1
        args = [parts, h,
                dinv,
                b[i].reshape(HID, 1),
                gamma[i].reshape(HID, 1),
                beta[i].reshape(HID, 1)]
        if prev is not None:
            args.append(prev)
        if has_next:
            args.append(Wh[i].T)
        res = _tc_layer_call(prev is not None, has_next)(*args)
        if has_next:
            out, h, hp = res
        else:
            out = res
        outs.append(out)
        prev = out

    emb_t = _tc_final_call(outs[0], outs[1], outs[2], outs[3],
                           lw.reshape(1, L))
    return emb_t.T


# speedup vs baseline: 29.4478x; 2.3349x over previous
import functools

import jax
import jax.numpy as jnp
from jax import lax
from jax.experimental import pallas as pl
from jax.experimental.pallas import tpu as pltpu
from jax.experimental.pallas import tpu_sc as plsc

N = 10000
E = 320000
HID = 20
L = 4

N_EG = 8
N_DG = 4
DPG = HID // N_DG
EPW = E // N_EG
CHUNK = 2000
NCHUNKS = EPW // CHUNK
NGROUPS = CHUNK // 16
TBL = N * DPG
TBL_PAD = TBL + 48

DEG_EPW = E // 32
DEG_GROUPS = DEG_EPW // 16


def _wid():
    return lax.axis_index("c") * 16 + lax.axis_index("s")


def _sc_mesh():
    return plsc.VectorSubcoreMesh(core_axis_name="c", subcore_axis_name="s")


UNROLL = 5


def _deg_body(col_hbm, ew_hbm, out_hbm, cbuf, wbuf, acc, sem):
    wid = _wid()
    base = wid * DEG_EPW
    pltpu.async_copy(col_hbm.at[pl.ds(base, DEG_EPW)], cbuf, sem)
    pltpu.async_copy(ew_hbm.at[pl.ds(base, DEG_EPW)], wbuf, sem)

    def zero(i, _):
        for u in range(4):
            acc[pl.ds(i * 64 + u * 16, 16)] = jnp.zeros((16,), jnp.float32)
        return 0

    lax.fori_loop(0, 10048 // 64, zero, 0)
    pltpu.make_async_copy(col_hbm.at[pl.ds(base, DEG_EPW)], cbuf, sem).wait()
    pltpu.make_async_copy(ew_hbm.at[pl.ds(base, DEG_EPW)], wbuf, sem).wait()

    def quint(g, _):
        occs = []
        mxq = jnp.int32(0)
        for u in range(UNROLL):
            off = g * (16 * UNROLL) + u * 16
            cc = cbuf[pl.ds(off, 16)]
            w = wbuf[pl.ds(off, 16)]
            occ, _last = plsc.scan_count(cc)
            plsc.addupdate_scatter(acc, [cc], w, mask=occ == 0)
            occs.append((cc, w, occ))
            mxq = jnp.maximum(mxq, jnp.max(occ))

        @pl.when(mxq > 0)
        def _():
            for cc, w, occ in occs:
                def dup_pass(k, _c):
                    plsc.addupdate_scatter(acc, [cc], w, mask=occ == k)
                    return 0

                lax.fori_loop(1, jnp.max(occ) + 1, dup_pass, 0)

        return 0

    lax.fori_loop(0, DEG_GROUPS // UNROLL, quint, 0)
    pltpu.sync_copy(acc.at[pl.ds(0, N)], out_hbm.at[pl.ds(wid * N, N)])


_deg_call = pl.kernel(
    _deg_body,
    out_type=jax.ShapeDtypeStruct((32 * N,), jnp.float32),
    mesh=_sc_mesh(),
    compiler_params=pltpu.CompilerParams(needs_layout_passes=False),
    scratch_types=[
        pltpu.VMEM((DEG_EPW,), jnp.int32),
        pltpu.VMEM((DEG_EPW,), jnp.float32),
        pltpu.VMEM((10048,), jnp.float32),
        pltpu.SemaphoreType.DMA,
    ],
)


def _edge_body(hp_hbm, row_hbm, col_hbm, ew_hbm, out_hbm,
               tbl, acc, rbuf0, cbuf0, wbuf0, rbuf1, cbuf1, wbuf1,
               tsem, sem0, sem1):
    wid = _wid()
    eg = wid // N_DG
    dg = wid % N_DG

    pltpu.async_copy(hp_hbm.at[pl.ds(dg * TBL, TBL)], tbl.at[pl.ds(0, TBL)],
                     tsem)

    def zero(i, _):
        for u in range(4):
            acc[pl.ds(i * 64 + u * 16, 16)] = jnp.zeros((16,), jnp.float32)
        return 0

    lax.fori_loop(0, TBL_PAD // 64, zero, 0)
    pltpu.make_async_copy(hp_hbm.at[pl.ds(dg * TBL, TBL)],
                          tbl.at[pl.ds(0, TBL)], tsem).wait()

    def start_chunk(idx, r, c, w, sem):
        base = eg * EPW + idx * CHUNK
        pltpu.async_copy(row_hbm.at[pl.ds(base, CHUNK)], r, sem)
        pltpu.async_copy(col_hbm.at[pl.ds(base, CHUNK)], c, sem)
        pltpu.async_copy(ew_hbm.at[pl.ds(base, CHUNK)], w, sem)

    def wait_chunk(r, c, w, sem):
        pltpu.make_async_copy(row_hbm.at[pl.ds(0, CHUNK)], r, sem).wait()
        pltpu.make_async_copy(col_hbm.at[pl.ds(0, CHUNK)], c, sem).wait()
        pltpu.make_async_copy(ew_hbm.at[pl.ds(0, CHUNK)], w, sem).wait()

    def process(rbuf, cbuf, wbuf):
        def quint(g, _c):
            groups = []
            for u in range(UNROLL):
                off = g * (16 * UNROLL) + u * 16
                r = rbuf[pl.ds(off, 16)]
                cc = cbuf[pl.ds(off, 16)]
                w = wbuf[pl.ds(off, 16)]
                occ, _last = plsc.scan_count(cc)
                groups.append((r, cc, w, occ))
            mxq = jnp.int32(0)
            for r, cc, w, occ in groups:
                first = occ == 0
                vs = [plsc.load_gather(tbl, [r + d * N]) * w
                      for d in range(DPG)]
                for d in range(DPG):
                    plsc.addupdate_scatter(acc, [cc + d * N], vs[d],
                                           mask=first)
                mxq = jnp.maximum(mxq, jnp.max(occ))

            @pl.when(mxq > 0)
            def _():
                for r, cc, w, occ in groups:
                    def dup_pass(k, _cc):
                        m = occ == k
                        dvs = [plsc.load_gather(tbl, [r + d * N]) * w
                               for d in range(DPG)]
                        for d in range(DPG):
                            plsc.addupdate_scatter(acc, [cc + d * N], dvs[d],
                                                   mask=m)
                        return 0

                    lax.fori_loop(1, jnp.max(occ) + 1, dup_pass, 0)

            return 0

        lax.fori_loop(0, NGROUPS // UNROLL, quint, 0)

    start_chunk(0, rbuf0, cbuf0, wbuf0, sem0)

    def two_chunks(ci, _):
        start_chunk(2 * ci + 1, rbuf1, cbuf1, wbuf1, sem1)
        wait_chunk(rbuf0, cbuf0, wbuf0, sem0)
        process(rbuf0, cbuf0, wbuf0)
        start_chunk((2 * ci + 2) % NCHUNKS, rbuf0, cbuf0, wbuf0, sem0)
        wait_chunk(rbuf1, cbuf1, wbuf1, sem1)
        process(rbuf1, cbuf1, wbuf1)
        return 0

    lax.fori_loop(0, NCHUNKS // 2, two_chunks, 0)
    wait_chunk(rbuf0, cbuf0, wbuf0, sem0)
    pltpu.sync_copy(acc.at[pl.ds(0, TBL)], out_hbm.at[pl.ds(wid * TBL, TBL)])


_edge_call = pl.kernel(
    _edge_body,
    out_type=jax.ShapeDtypeStruct((N_EG * N_DG * TBL,), jnp.float32),
    mesh=_sc_mesh(),
    compiler_params=pltpu.CompilerParams(needs_layout_passes=False),
    scratch_types=[
        pltpu.VMEM((TBL_PAD,), jnp.float32),
        pltpu.VMEM((TBL_PAD,), jnp.float32),
        pltpu.VMEM((CHUNK,), jnp.int32),
        pltpu.VMEM((CHUNK,), jnp.int32),
        pltpu.VMEM((CHUNK,), jnp.float32),
        pltpu.VMEM((CHUNK,), jnp.int32),
        pltpu.VMEM((CHUNK,), jnp.int32),
        pltpu.VMEM((CHUNK,), jnp.float32),
        pltpu.SemaphoreType.DMA,
        pltpu.SemaphoreType.DMA,
        pltpu.SemaphoreType.DMA,
    ],
)


def _tc1_body(deg_parts_ref, ft_ref, w0t_ref, dinv_ref, h0_ref, hp0_ref):
    deg = jnp.sum(deg_parts_ref[...], axis=0, keepdims=True) + 1.0
    dinv = 1.0 / jnp.sqrt(deg)
    dinv_ref[...] = dinv
    h0 = jnp.dot(w0t_ref[...], ft_ref[...], preferred_element_type=jnp.float32)
    h0_ref[...] = h0
    hp0_ref[...] = h0 * dinv


_tc1_call = pl.pallas_call(
    _tc1_body,
    out_shape=(
        jax.ShapeDtypeStruct((1, N), jnp.float32),
        jax.ShapeDtypeStruct((HID, N), jnp.float32),
        jax.ShapeDtypeStruct((HID, N), jnp.float32),
    ),
)


def _tc_layer_body(has_prev, has_next,
                   parts_ref, h_ref, dinv_ref, bias_ref, gam_ref, bta_ref,
                   *rest):
    rest = list(rest)
    prev_ref = rest.pop(0) if has_prev else None
    wt_ref = rest.pop(0) if has_next else None
    out_ref = rest.pop(0)
    h_next_ref = rest.pop(0) if has_next else None
    hp_next_ref = rest.pop(0) if has_next else None

    acc = parts_ref[0]
    for k in range(1, N_EG):
        acc = acc + parts_ref[k]
    dinv = dinv_ref[...]
    agg = acc * dinv + h_ref[...] * (dinv * dinv) + bias_ref[...]
    m = jnp.mean(agg, axis=1, keepdims=True)
    cen = agg - m
    var = jnp.mean(cen * cen, axis=1, keepdims=True)
    x = cen / jnp.sqrt(var + 1e-5) * gam_ref[...] + bta_ref[...]
    x = jnp.maximum(x, 0.0)
    if has_prev:
        x = x + 0.7 * prev_ref[...]
    out_ref[...] = x
    if has_next:
        h_next = jnp.dot(wt_ref[...], x, preferred_element_type=jnp.float32)
        h_next_ref[...] = h_next
        hp_next_ref[...] = h_next * dinv


def _tc_layer_call(has_prev, has_next):
    n_out = 1 + (2 if has_next else 0)
    shapes = [jax.ShapeDtypeStruct((HID, N), jnp.float32)] * n_out
    return pl.pallas_call(
        functools.partial(_tc_layer_body, has_prev, has_next),
        out_shape=tuple(shapes) if n_out > 1 else shapes[0],
    )


def _tc_final_body(out0_ref, out1_ref, out2_ref, out3_ref, lw_ref, emb_ref):
    w = jax.nn.softmax(lw_ref[...], axis=-1)
    emb = (out0_ref[...] * w[0, 0] + out1_ref[...] * w[0, 1]
           + out2_ref[...] * w[0, 2] + out3_ref[...] * w[0, 3])
    emb_ref[...] = emb


_tc_final_call = pl.pallas_call(
    _tc_final_body,
    out_shape=jax.ShapeDtypeStruct((HID, N), jnp.float32),
)


def kernel(features, edges, edge_weight, W0, Wh, b, gamma, beta, lw):
    row = edges[0]
    col = edges[1]
    ft = features.T
    w0t = W0.T

    deg_parts = _deg_call(col, edge_weight).reshape(32, N)
    dinv, h, hp = _tc1_call(deg_parts, ft, w0t)

    outs = []
    prev = None
    for i in range(L):
        parts = _edge_call(hp.reshape(HID * N), row, col, edge_weight)
        parts = parts.reshape(N_EG, HID, N)
        has_next = i < L - 1
        args = [parts, h,
                dinv,
                b[i].reshape(HID, 1),
                gamma[i].reshape(HID, 1),
                beta[i].reshape(HID, 1)]
        if prev is not None:
            args.append(prev)
        if has_next:
            args.append(Wh[i].T)
        res = _tc_layer_call(prev is not None, has_next)(*args)
        if has_next:
            out, h, hp = res
        else:
            out = res
        outs.append(out)
        prev = out

    emb_t = _tc_final_call(outs[0], outs[1], outs[2], outs[3],
                           lw.reshape(1, L))
    return emb_t.T


# speedup vs baseline: 29.5693x; 1.0041x over previous
import functools

import jax
import jax.numpy as jnp
from jax import lax
from jax.experimental import pallas as pl
from jax.experimental.pallas import tpu as pltpu
from jax.experimental.pallas import tpu_sc as plsc

N = 10000
E = 320000
HID = 20
L = 4

N_EG = 8
N_DG = 4
DPG = HID // N_DG
EPW = E // N_EG
CHUNK = 4000
NCHUNKS = EPW // CHUNK
NGROUPS = CHUNK // 16
TBL = N * DPG
TBL_PAD = TBL + 48

DEG_EPW = E // 32
DEG_GROUPS = DEG_EPW // 16


def _wid():
    return lax.axis_index("c") * 16 + lax.axis_index("s")


def _sc_mesh():
    return plsc.VectorSubcoreMesh(core_axis_name="c", subcore_axis_name="s")


UNROLL = 5


def _deg_body(col_hbm, ew_hbm, out_hbm, cbuf, wbuf, acc, sem):
    wid = _wid()
    base = wid * DEG_EPW
    pltpu.async_copy(col_hbm.at[pl.ds(base, DEG_EPW)], cbuf, sem)
    pltpu.async_copy(ew_hbm.at[pl.ds(base, DEG_EPW)], wbuf, sem)

    def zero(i, _):
        for u in range(4):
            acc[pl.ds(i * 64 + u * 16, 16)] = jnp.zeros((16,), jnp.float32)
        return 0

    lax.fori_loop(0, 10048 // 64, zero, 0)
    pltpu.make_async_copy(col_hbm.at[pl.ds(base, DEG_EPW)], cbuf, sem).wait()
    pltpu.make_async_copy(ew_hbm.at[pl.ds(base, DEG_EPW)], wbuf, sem).wait()

    def quint(g, _):
        occs = []
        mxq = jnp.int32(0)
        for u in range(UNROLL):
            off = g * (16 * UNROLL) + u * 16
            cc = cbuf[pl.ds(off, 16)]
            w = wbuf[pl.ds(off, 16)]
            occ, _last = plsc.scan_count(cc)
            plsc.addupdate_scatter(acc, [cc], w, mask=occ == 0)
            occs.append((cc, w, occ))
            mxq = jnp.maximum(mxq, jnp.max(occ))

        @pl.when(mxq > 0)
        def _():
            for cc, w, occ in occs:
                def dup_pass(k, _c):
                    plsc.addupdate_scatter(acc, [cc], w, mask=occ == k)
                    return 0

                lax.fori_loop(1, jnp.max(occ) + 1, dup_pass, 0)

        return 0

    lax.fori_loop(0, DEG_GROUPS // UNROLL, quint, 0)
    pltpu.sync_copy(acc.at[pl.ds(0, N)], out_hbm.at[pl.ds(wid * N, N)])


_deg_call = pl.kernel(
    _deg_body,
    out_type=jax.ShapeDtypeStruct((32 * N,), jnp.float32),
    mesh=_sc_mesh(),
    compiler_params=pltpu.CompilerParams(needs_layout_passes=False,
                                         disable_bounds_checks=True),
    scratch_types=[
        pltpu.VMEM((DEG_EPW,), jnp.int32),
        pltpu.VMEM((DEG_EPW,), jnp.float32),
        pltpu.VMEM((10048,), jnp.float32),
        pltpu.SemaphoreType.DMA,
    ],
)


def _edge_body(hp_hbm, row_hbm, col_hbm, ew_hbm, out_hbm,
               tbl, acc, rbuf0, cbuf0, wbuf0, rbuf1, cbuf1, wbuf1,
               tsem, sem0, sem1):
    wid = _wid()
    eg = wid // N_DG
    dg = wid % N_DG

    pltpu.async_copy(hp_hbm.at[pl.ds(dg * TBL, TBL)], tbl.at[pl.ds(0, TBL)],
                     tsem)

    def zero(i, _):
        for u in range(4):
            acc[pl.ds(i * 64 + u * 16, 16)] = jnp.zeros((16,), jnp.float32)
        return 0

    lax.fori_loop(0, TBL_PAD // 64, zero, 0)
    pltpu.make_async_copy(hp_hbm.at[pl.ds(dg * TBL, TBL)],
                          tbl.at[pl.ds(0, TBL)], tsem).wait()

    def start_chunk(idx, r, c, w, sem):
        base = eg * EPW + idx * CHUNK
        pltpu.async_copy(row_hbm.at[pl.ds(base, CHUNK)], r, sem)
        pltpu.async_copy(col_hbm.at[pl.ds(base, CHUNK)], c, sem)
        pltpu.async_copy(ew_hbm.at[pl.ds(base, CHUNK)], w, sem)

    def wait_chunk(r, c, w, sem):
        pltpu.make_async_copy(row_hbm.at[pl.ds(0, CHUNK)], r, sem).wait()
        pltpu.make_async_copy(col_hbm.at[pl.ds(0, CHUNK)], c, sem).wait()
        pltpu.make_async_copy(ew_hbm.at[pl.ds(0, CHUNK)], w, sem).wait()

    def process(rbuf, cbuf, wbuf):
        def quint(g, _c):
            groups = []
            for u in range(UNROLL):
                off = g * (16 * UNROLL) + u * 16
                r = rbuf[pl.ds(off, 16)]
                cc = cbuf[pl.ds(off, 16)]
                w = wbuf[pl.ds(off, 16)]
                occ, _last = plsc.scan_count(cc)
                groups.append((r, cc, w, occ))
            mxq = jnp.int32(0)
            for r, cc, w, occ in groups:
                first = occ == 0
                vs = [plsc.load_gather(tbl, [r + d * N]) * w
                      for d in range(DPG)]
                for d in range(DPG):
                    plsc.addupdate_scatter(acc, [cc + d * N], vs[d],
                                           mask=first)
                mxq = jnp.maximum(mxq, jnp.max(occ))

            @pl.when(mxq > 0)
            def _():
                for r, cc, w, occ in groups:
                    def dup_pass(k, _cc):
                        m = occ == k
                        dvs = [plsc.load_gather(tbl, [r + d * N]) * w
                               for d in range(DPG)]
                        for d in range(DPG):
                            plsc.addupdate_scatter(acc, [cc + d * N], dvs[d],
                                                   mask=m)
                        return 0

                    lax.fori_loop(1, jnp.max(occ) + 1, dup_pass, 0)

            return 0

        lax.fori_loop(0, NGROUPS // UNROLL, quint, 0)

    start_chunk(0, rbuf0, cbuf0, wbuf0, sem0)

    def two_chunks(ci, _):
        start_chunk(2 * ci + 1, rbuf1, cbuf1, wbuf1, sem1)
        wait_chunk(rbuf0, cbuf0, wbuf0, sem0)
        process(rbuf0, cbuf0, wbuf0)
        start_chunk((2 * ci + 2) % NCHUNKS, rbuf0, cbuf0, wbuf0, sem0)
        wait_chunk(rbuf1, cbuf1, wbuf1, sem1)
        process(rbuf1, cbuf1, wbuf1)
        return 0

    lax.fori_loop(0, NCHUNKS // 2, two_chunks, 0)
    wait_chunk(rbuf0, cbuf0, wbuf0, sem0)
    pltpu.sync_copy(acc.at[pl.ds(0, TBL)], out_hbm.at[pl.ds(wid * TBL, TBL)])


_edge_call = pl.kernel(
    _edge_body,
    out_type=jax.ShapeDtypeStruct((N_EG * N_DG * TBL,), jnp.float32),
    mesh=_sc_mesh(),
    compiler_params=pltpu.CompilerParams(needs_layout_passes=False,
                                         disable_bounds_checks=True),
    scratch_types=[
        pltpu.VMEM((TBL_PAD,), jnp.float32),
        pltpu.VMEM((TBL_PAD,), jnp.float32),
        pltpu.VMEM((CHUNK,), jnp.int32),
        pltpu.VMEM((CHUNK,), jnp.int32),
        pltpu.VMEM((CHUNK,), jnp.float32),
        pltpu.VMEM((CHUNK,), jnp.int32),
        pltpu.VMEM((CHUNK,), jnp.int32),
        pltpu.VMEM((CHUNK,), jnp.float32),
        pltpu.SemaphoreType.DMA,
        pltpu.SemaphoreType.DMA,
        pltpu.SemaphoreType.DMA,
    ],
)


def _tc0_body(f_ref, w0_ref, h0_ref):
    h0_ref[...] = lax.dot_general(w0_ref[...], f_ref[...],
                                  (((0,), (1,)), ((), ())),
                                  preferred_element_type=jnp.float32)


_tc0_call = pl.pallas_call(
    _tc0_body,
    out_shape=jax.ShapeDtypeStruct((HID, N), jnp.float32),
)


def _tc1b_body(deg_parts_ref, h0_ref, dinv_ref, hp0_ref):
    deg = jnp.sum(deg_parts_ref[...], axis=0, keepdims=True) + 1.0
    dinv = 1.0 / jnp.sqrt(deg)
    dinv_ref[...] = dinv
    hp0_ref[...] = h0_ref[...] * dinv


_tc1b_call = pl.pallas_call(
    _tc1b_body,
    out_shape=(
        jax.ShapeDtypeStruct((1, N), jnp.float32),
        jax.ShapeDtypeStruct((HID, N), jnp.float32),
    ),
)


def _tc_mid_body(has_prev,
                 parts_ref, h_ref, dinv_ref, bias_ref, gam_ref, bta_ref,
                 *rest):
    rest = list(rest)
    prev_ref = rest.pop(0) if has_prev else None
    wt_ref, out_ref, h_next_ref, hp_next_ref = rest

    acc = parts_ref[0]
    for k in range(1, N_EG):
        acc = acc + parts_ref[k]
    dinv = dinv_ref[...]
    agg = acc * dinv + h_ref[...] * (dinv * dinv) + bias_ref[...]
    m = jnp.mean(agg, axis=1, keepdims=True)
    cen = agg - m
    var = jnp.mean(cen * cen, axis=1, keepdims=True)
    x = cen / jnp.sqrt(var + 1e-5) * gam_ref[...] + bta_ref[...]
    x = jnp.maximum(x, 0.0)
    if has_prev:
        x = x + 0.7 * prev_ref[...]
    out_ref[...] = x
    h_next = jnp.dot(wt_ref[...], x, preferred_element_type=jnp.float32)
    h_next_ref[...] = h_next
    hp_next_ref[...] = h_next * dinv


def _tc_mid_call(has_prev):
    return pl.pallas_call(
        functools.partial(_tc_mid_body, has_prev),
        out_shape=(
            jax.ShapeDtypeStruct((HID, N), jnp.float32),
            jax.ShapeDtypeStruct((HID, N), jnp.float32),
            jax.ShapeDtypeStruct((HID, N), jnp.float32),
        ),
    )


def _tc_last_body(parts_ref, h_ref, dinv_ref, bias_ref, gam_ref, bta_ref,
                  prev_ref, out0_ref, out1_ref, out2_ref, lw_ref, emb_ref):
    acc = parts_ref[0]
    for k in range(1, N_EG):
        acc = acc + parts_ref[k]
    dinv = dinv_ref[...]
    agg = acc * dinv + h_ref[...] * (dinv * dinv) + bias_ref[...]
    m = jnp.mean(agg, axis=1, keepdims=True)
    cen = agg - m
    var = jnp.mean(cen * cen, axis=1, keepdims=True)
    x = cen / jnp.sqrt(var + 1e-5) * gam_ref[...] + bta_ref[...]
    x = jnp.maximum(x, 0.0) + 0.7 * prev_ref[...]
    w = jax.nn.softmax(lw_ref[...], axis=-1)
    emb_ref[...] = (out0_ref[...] * w[0, 0] + out1_ref[...] * w[0, 1]
                    + out2_ref[...] * w[0, 2] + x * w[0, 3])


_tc_last_call = pl.pallas_call(
    _tc_last_body,
    out_shape=jax.ShapeDtypeStruct((HID, N), jnp.float32),
)


def kernel(features, edges, edge_weight, W0, Wh, b, gamma, beta, lw):
    row = edges[0]
    col = edges[1]

    deg_parts = _deg_call(col, edge_weight).reshape(32, N)
    h = _tc0_call(features, W0)
    dinv, hp = _tc1b_call(deg_parts, h)

    outs = []
    prev = None
    for i in range(L - 1):
        parts = _edge_call(hp.reshape(HID * N), row, col, edge_weight)
        parts = parts.reshape(N_EG, HID, N)
        args = [parts, h,
                dinv,
                b[i].reshape(HID, 1),
                gamma[i].reshape(HID, 1),
                beta[i].reshape(HID, 1)]
        if prev is not None:
            args.append(prev)
        args.append(Wh[i].T)
        out, h, hp = _tc_mid_call(prev is not None)(*args)
        outs.append(out)
        prev = out

    parts = _edge_call(hp.reshape(HID * N), row, col, edge_weight)
    parts = parts.reshape(N_EG, HID, N)
    emb_t = _tc_last_call(parts, h, dinv,
                          b[3].reshape(HID, 1), gamma[3].reshape(HID, 1),
                          beta[3].reshape(HID, 1), prev,
                          outs[0], outs[1], outs[2], lw.reshape(1, L))
    return emb_t.T
